# baseline (speedup 1.0000x reference)
import jax
import jax.numpy as jnp
import numpy as np
from jax.experimental import pallas as pl
from jax.experimental.pallas import tpu as pltpu

_NB = 50
_H = 33
_OD = 16
_FP = 40
_GT = 64
_SUB = 8
_LANE = 512
_BCOL = 2048

_F32 = jnp.float32
_INV_SQRT2 = np.float32(1.0 / np.sqrt(2.0))


def _tab_kernel(embdT_ref, w1lo_ref, w1hi_ref, b1_ref, w2_ref,
                ttab_ref, w2rep_ref):
    e0 = embdT_ref[:, 0:50]
    e1 = embdT_ref[:, 1:51]
    tt = (
        jnp.dot(w1lo_ref[...], e0, preferred_element_type=_F32)
        + jnp.dot(w1hi_ref[...], e1, preferred_element_type=_F32)
        + b1_ref[...]
    )
    pad = jnp.pad(tt, ((0, _FP - _H), (0, _GT - _NB)))
    ttab_ref[...] = jnp.broadcast_to(pad[:, None, :], (_FP, _SUB, _GT))
    w2rep_ref[...] = jnp.broadcast_to(
        (0.5 * w2_ref[...])[:, :, None, None], (_OD, _H, _SUB, _LANE)
    )


def _take(tab, idx):
    return jnp.take_along_axis(tab, idx, axis=1, mode="promise_in_bounds")


def _main_kernel(x_ref, ttab_ref, w2rep_ref, w1d_ref, b2_ref,
                 oob_ref, out_ref, g_scr):
    nr = x_ref.shape[0] // _SUB
    nc = x_ref.shape[1] // _LANE
    for r in range(nr):
        for c in range(nc):
            par = (r * nc + c) % 2
            r0 = r * _SUB
            c0 = c * _LANE
            x = x_ref[r0:r0 + _SUB, c0:c0 + _LANE]

            step = np.float32(1.0) / np.float32(_NB - 1)
            k0f = jnp.clip(jnp.floor(x * np.float32(_NB - 1)), 0.0,
                           np.float32(_NB - 2))
            bkm1 = (k0f - 1.0) * step
            bk0 = k0f * step
            bk1 = (k0f + 1.0) * step
            bk2 = (k0f + 2.0) * step
            c0m = bk0 < x
            c1m = bk1 < x
            idxf = k0f + c0m.astype(_F32) + c1m.astype(_F32)
            gi = jnp.clip(idxf, 1.0, np.float32(_NB - 1)).astype(jnp.int32)
            lower = jnp.where(c1m, bk1, jnp.where(c0m, bk0, bkm1))
            higher = jnp.where(c1m, bk2, jnp.where(c0m, bk1, bk0))
            obl = idxf < 0.5
            obh = idxf > np.float32(_NB - 0.5)
            inb = jnp.logical_not(jnp.logical_or(obl, obh))
            d = jnp.where(inb, (x - lower) / (higher - lower), 0.0)

            for j in range(_H):
                pj = _take(ttab_ref[j], gi)
                hj = pj + d * w1d_ref[j]
                g_scr[par, j] = hj * (1.0 + jax.lax.erf(hj * _INV_SQRT2))

            for i0 in range(0, _OD, 4):
                outs = [jnp.full((_SUB, _LANE), b2_ref[i0 + i])
                        for i in range(4)]
                for j in range(_H):
                    gj = g_scr[par, j]
                    for i in range(4):
                        outs[i] += gj * w2rep_ref[i0 + i, j]
                for i in range(4):
                    o = jnp.where(obl, oob_ref[0, i0 + i], outs[i])
                    o = jnp.where(obh, oob_ref[1, i0 + i], o)
                    out_ref[i0 + i, r0:r0 + _SUB, c0:c0 + _LANE] = o


def kernel(x, embd, embd_out_of_bounds, W1, b1, W2, b2):
    B, N, _ = x.shape
    nblk = N // _BCOL
    ttab, w2rep = pl.pallas_call(
        _tab_kernel,
        out_shape=(
            jax.ShapeDtypeStruct((_FP, _SUB, _GT), _F32),
            jax.ShapeDtypeStruct((_OD, _H, _SUB, _LANE), _F32),
        ),
    )(embd.T, W1[:, 0:16], W1[:, 16:32], b1.reshape(_H, 1), W2)

    x2 = x.reshape(B, N)
    parts = []
    for blk in range(nblk):
        outp = pl.pallas_call(
            _main_kernel,
            grid=(1,),
            in_specs=[
                pl.BlockSpec((B, _BCOL), lambda i: (0, 0)),
                pl.BlockSpec((_FP, _SUB, _GT), lambda i: (0, 0, 0)),
                pl.BlockSpec((_OD, _H, _SUB, _LANE),
                             lambda i: (0, 0, 0, 0)),
                pl.BlockSpec(memory_space=pltpu.SMEM),
                pl.BlockSpec(memory_space=pltpu.SMEM),
                pl.BlockSpec(memory_space=pltpu.SMEM),
            ],
            out_specs=pl.BlockSpec((_OD, B, _BCOL), lambda i: (0, 0, 0)),
            out_shape=jax.ShapeDtypeStruct((_OD, B, _BCOL), _F32),
            scratch_shapes=[pltpu.VMEM((2, _H, _SUB, _LANE), _F32)],
        )(x2[:, blk * _BCOL:(blk + 1) * _BCOL], ttab, w2rep, W1[:, 32],
          b2, embd_out_of_bounds)
        parts.append(outp.transpose(1, 2, 0))
    return jnp.concatenate(parts, axis=1)


# speedup vs baseline: 1.4627x; 1.4627x over previous
import jax
import jax.numpy as jnp
import numpy as np
from jax.experimental import pallas as pl
from jax.experimental.pallas import tpu as pltpu

_NB = 50
_H = 33
_OD = 16
_FP = 40
_GT = 64
_SUB = 8
_LANE = 512
_BCOL = 2048

_F32 = jnp.float32
_INV_SQRT2 = np.float32(1.0 / np.sqrt(2.0))


def _take(tab, idx):
    return jnp.take_along_axis(tab, idx, axis=1, mode="promise_in_bounds")


def _main_kernel(x_ref, embdT_ref, w1lo_ref, w1hi_ref, b1_ref, w2_ref,
                 w1d_ref, b2_ref, oob_ref, out_ref,
                 g_scr, ttab_scr, w2rep_scr):
    e0 = embdT_ref[:, 0:50]
    e1 = embdT_ref[:, 1:51]
    tt = (
        jnp.dot(w1lo_ref[...], e0, preferred_element_type=_F32)
        + jnp.dot(w1hi_ref[...], e1, preferred_element_type=_F32)
        + b1_ref[...]
    )
    pad = jnp.pad(tt, ((0, _FP - _H), (0, _GT - _NB)))
    ttab_scr[...] = jnp.broadcast_to(pad[:, None, :], (_FP, _SUB, _GT))
    w2rep_scr[...] = jnp.broadcast_to(
        (0.5 * w2_ref[...])[:, :, None, None], (_OD, _H, _SUB, _LANE)
    )

    nr = x_ref.shape[0] // _SUB
    nc = x_ref.shape[1] // _LANE
    for r in range(nr):
        for c in range(nc):
            par = (r * nc + c) % 2
            r0 = r * _SUB
            c0 = c * _LANE
            x = x_ref[r0:r0 + _SUB, c0:c0 + _LANE]

            step = np.float32(1.0) / np.float32(_NB - 1)
            k0f = jnp.clip(jnp.floor(x * np.float32(_NB - 1)), 0.0,
                           np.float32(_NB - 2))
            bkm1 = (k0f - 1.0) * step
            bk0 = k0f * step
            bk1 = (k0f + 1.0) * step
            bk2 = (k0f + 2.0) * step
            c0m = bk0 < x
            c1m = bk1 < x
            idxf = k0f + c0m.astype(_F32) + c1m.astype(_F32)
            gi = jnp.clip(idxf, 1.0, np.float32(_NB - 1)).astype(jnp.int32)
            lower = jnp.where(c1m, bk1, jnp.where(c0m, bk0, bkm1))
            higher = jnp.where(c1m, bk2, jnp.where(c0m, bk1, bk0))
            obl = idxf < 0.5
            obh = idxf > np.float32(_NB - 0.5)
            inb = jnp.logical_not(jnp.logical_or(obl, obh))
            d = jnp.where(inb, (x - lower) / (higher - lower), 0.0)

            for j in range(_H):
                pj = _take(ttab_scr[j], gi)
                hj = pj + d * w1d_ref[j]
                g_scr[par, j] = hj * (1.0 + jax.lax.erf(hj * _INV_SQRT2))

            for i0 in range(0, _OD, 4):
                outs = [jnp.full((_SUB, _LANE), b2_ref[i0 + i])
                        for i in range(4)]
                for j in range(_H):
                    gj = g_scr[par, j]
                    for i in range(4):
                        outs[i] += gj * w2rep_scr[i0 + i, j]
                for i in range(4):
                    o = jnp.where(obl, oob_ref[0, i0 + i], outs[i])
                    o = jnp.where(obh, oob_ref[1, i0 + i], o)
                    out_ref[i0 + i, r0:r0 + _SUB, c0:c0 + _LANE] = o


def kernel(x, embd, embd_out_of_bounds, W1, b1, W2, b2):
    B, N, _ = x.shape
    nblk = N // _BCOL
    x2 = x.reshape(B, N)
    outp = pl.pallas_call(
        _main_kernel,
        grid=(nblk,),
        in_specs=[
            pl.BlockSpec((B, _BCOL), lambda i: (0, i)),
            pl.BlockSpec((16, 51), lambda i: (0, 0)),
            pl.BlockSpec((_H, 16), lambda i: (0, 0)),
            pl.BlockSpec((_H, 16), lambda i: (0, 0)),
            pl.BlockSpec((_H, 1), lambda i: (0, 0)),
            pl.BlockSpec((_OD, _H), lambda i: (0, 0)),
            pl.BlockSpec(memory_space=pltpu.SMEM),
            pl.BlockSpec(memory_space=pltpu.SMEM),
            pl.BlockSpec(memory_space=pltpu.SMEM),
        ],
        out_specs=pl.BlockSpec((_OD, B, _BCOL), lambda i: (0, 0, i)),
        out_shape=jax.ShapeDtypeStruct((_OD, B, N), _F32),
        scratch_shapes=[
            pltpu.VMEM((2, _H, _SUB, _LANE), _F32),
            pltpu.VMEM((_FP, _SUB, _GT), _F32),
            pltpu.VMEM((_OD, _H, _SUB, _LANE), _F32),
        ],
        compiler_params=pltpu.CompilerParams(
            dimension_semantics=("parallel",),
        ),
    )(x2, embd.T, W1[:, 0:16], W1[:, 16:32], b1.reshape(_H, 1), W2,
      W1[:, 32], b2, embd_out_of_bounds)

    return outp.transpose(1, 2, 0)


# speedup vs baseline: 1.4662x; 1.0024x over previous
import jax
import jax.numpy as jnp
import numpy as np
from jax.experimental import pallas as pl
from jax.experimental.pallas import tpu as pltpu

_NB = 50
_H = 33
_OD = 16
_FP = 40
_GT = 64
_SUB = 8
_LANE = 512
_BCOL = 2048

_F32 = jnp.float32
_INV_SQRT2 = np.float32(1.0 / np.sqrt(2.0))


def _take(tab, idx):
    return jnp.take_along_axis(tab, idx, axis=1, mode="promise_in_bounds")


def _main_kernel(x_ref, embdT_ref, w1lo_ref, w1hi_ref, b1_ref, w2_ref,
                 w1d_ref, b2_ref, oob_ref, out_ref,
                 g_scr, ttab_scr, w2rep_scr):
    e0 = embdT_ref[:, 0:50]
    e1 = embdT_ref[:, 1:51]
    tt = (
        jnp.dot(w1lo_ref[...], e0, preferred_element_type=_F32)
        + jnp.dot(w1hi_ref[...], e1, preferred_element_type=_F32)
        + b1_ref[...]
    )
    pad = jnp.pad(tt, ((0, _FP - _H), (0, _GT - _NB)))
    ttab_scr[...] = jnp.broadcast_to(pad[:, None, :], (_FP, _SUB, _GT))
    w2rep_scr[...] = jnp.broadcast_to(
        (0.5 * w2_ref[...])[:, :, None, None], (_OD, _H, _SUB, _LANE)
    )

    nr = x_ref.shape[0] // _SUB
    nc = x_ref.shape[1] // _LANE
    for r in range(nr):
        for c in range(nc):
            par = (r * nc + c) % 2
            r0 = r * _SUB
            c0 = c * _LANE
            x = x_ref[r0:r0 + _SUB, c0:c0 + _LANE]

            step = np.float32(1.0) / np.float32(_NB - 1)
            k0f = jnp.clip(jnp.floor(x * np.float32(_NB - 1)), 0.0,
                           np.float32(_NB - 2))
            bkm1 = (k0f - 1.0) * step
            bk0 = k0f * step
            bk1 = (k0f + 1.0) * step
            bk2 = (k0f + 2.0) * step
            c0m = bk0 < x
            c1m = bk1 < x
            idxf = k0f + c0m.astype(_F32) + c1m.astype(_F32)
            gi = jnp.clip(idxf, 1.0, np.float32(_NB - 1)).astype(jnp.int32)
            lower = jnp.where(c1m, bk1, jnp.where(c0m, bk0, bkm1))
            higher = jnp.where(c1m, bk2, jnp.where(c0m, bk1, bk0))
            obl = idxf < 0.5
            obh = idxf > np.float32(_NB - 0.5)
            inb = jnp.logical_not(jnp.logical_or(obl, obh))
            d = jnp.where(inb, (x - lower) / (higher - lower), 0.0)

            for j in range(_H):
                pj = _take(ttab_scr[j], gi)
                hj = pj + d * w1d_ref[j]
                g_scr[par, j] = hj * (1.0 + jax.lax.erf(hj * _INV_SQRT2))

            for i0 in range(0, _OD, 8):
                outs = [jnp.full((_SUB, _LANE), b2_ref[i0 + i])
                        for i in range(8)]
                for j in range(_H):
                    gj = g_scr[par, j]
                    for i in range(8):
                        outs[i] += gj * w2rep_scr[i0 + i, j]
                for i in range(8):
                    o = jnp.where(obl, oob_ref[0, i0 + i], outs[i])
                    o = jnp.where(obh, oob_ref[1, i0 + i], o)
                    out_ref[i0 + i, r0:r0 + _SUB, c0:c0 + _LANE] = o


def kernel(x, embd, embd_out_of_bounds, W1, b1, W2, b2):
    B, N, _ = x.shape
    nblk = N // _BCOL
    x2 = x.reshape(B, N)
    outp = pl.pallas_call(
        _main_kernel,
        grid=(nblk,),
        in_specs=[
            pl.BlockSpec((B, _BCOL), lambda i: (0, i)),
            pl.BlockSpec((16, 51), lambda i: (0, 0)),
            pl.BlockSpec((_H, 16), lambda i: (0, 0)),
            pl.BlockSpec((_H, 16), lambda i: (0, 0)),
            pl.BlockSpec((_H, 1), lambda i: (0, 0)),
            pl.BlockSpec((_OD, _H), lambda i: (0, 0)),
            pl.BlockSpec(memory_space=pltpu.SMEM),
            pl.BlockSpec(memory_space=pltpu.SMEM),
            pl.BlockSpec(memory_space=pltpu.SMEM),
        ],
        out_specs=pl.BlockSpec((_OD, B, _BCOL), lambda i: (0, 0, i)),
        out_shape=jax.ShapeDtypeStruct((_OD, B, N), _F32),
        scratch_shapes=[
            pltpu.VMEM((2, _H, _SUB, _LANE), _F32),
            pltpu.VMEM((_FP, _SUB, _GT), _F32),
            pltpu.VMEM((_OD, _H, _SUB, _LANE), _F32),
        ],
        compiler_params=pltpu.CompilerParams(
            dimension_semantics=("parallel",),
        ),
    )(x2, embd.T, W1[:, 0:16], W1[:, 16:32], b1.reshape(_H, 1), W2,
      W1[:, 32], b2, embd_out_of_bounds)

    return outp.transpose(1, 2, 0)
